# Initial kernel scaffold; baseline (speedup 1.0000x reference)
import jax
import jax.numpy as jnp
from jax.experimental import pallas as pl


def kernel(x, W_lin, b_lin):
    raise NotImplementedError("write your pallas kernel here")


# speedup vs baseline: 112.7867x; 112.7867x over previous
import functools

import numpy as np
import jax
import jax.numpy as jnp
from jax import lax
from jax.experimental import pallas as pl
from jax.experimental.pallas import tpu as pltpu
from jax.experimental.pallas import tpu_sc as plsc

B = 4096
DIN = 128
DOUT = 112
NSPL = 16
NT = 50
SB = 256
CANVAS_W = 1024
NCORES = 2
NSUB = 16
NWORK = NCORES * NSUB
SPT = B // NWORK
CHUNK = 16
NCHUNKS = SPT // CHUNK

_LOG2PI = float(np.log(2.0 * np.pi))
ENTROPY_C = float(DOUT * (0.5 + 0.5 * _LOG2PI))
LOGPROB_C = float(DOUT * (-0.5 * _LOG2PI))


def _tc_points_kernel(w_ref, b_ref, x_ref, b0_ref, b1_ref, b2_ref,
                      sample_ref, idx_ref, wgt_ref):
    xb = x_ref[...].astype(jnp.bfloat16)
    mu = lax.dot_general(w_ref[...].astype(jnp.bfloat16), xb,
                         (((1,), (1,)), ((), ())),
                         preferred_element_type=jnp.float32)
    mu = mu + b_ref[...]
    sg = 1.0 / (1.0 + jnp.exp(-mu))
    sample_ref[...] = sg[0:DOUT]
    par = sg[DOUT:2 * DOUT] * 28.0
    p0x = par[0:16]
    p0y = par[16:32]
    p1x = par[32:48]
    p1y = par[48:64]
    p2x = par[64:80]
    p2y = par[80:96]
    wgt_ref[...] = par[96:112] * (-0.003)
    b0 = b0_ref[...]
    b1 = b1_ref[...]
    b2 = b2_ref[...]
    px = (b0 * p0x[:, None, :] + b1 * p1x[:, None, :]) + b2 * p2x[:, None, :]
    py = (b0 * p0y[:, None, :] + b1 * p1y[:, None, :]) + b2 * p2y[:, None, :]
    cx = jnp.round(px)
    cy = jnp.round(py)
    idx_ref[...] = (cx * 32.0 + cy).astype(jnp.int32)


def _tc_fold_kernel(e_ref, sk_ref, lp_ref, en_ref):
    e = e_ref[...]
    ly = jnp.concatenate([e[:, :, 0:1], e[:, :, 0:27]], axis=2)
    ry = jnp.concatenate([e[:, :, 1:28],
                          e[:, :, 27:28] + 3.0 * e[:, :, 28:29]], axis=2)
    ty = (ly + e[:, :, 0:28]) + ry
    lx = jnp.concatenate([ty[:, 0:1], ty[:, 0:27]], axis=1)
    rx = jnp.concatenate([ty[:, 1:28],
                          ty[:, 27:28] + 3.0 * ty[:, 28:29]], axis=1)
    tx = (lx + ty[:, 0:28]) + rx
    sk_ref[...] = jnp.clip(tx + 0.3, 0.0, 1.0)
    lp_ref[...] = jnp.full((SB,), LOGPROB_C, jnp.float32)
    en_ref[...] = jnp.full((SB,), ENTROPY_C, jnp.float32)


def _make_sc_scatter():
    mesh = plsc.VectorSubcoreMesh(core_axis_name="c", subcore_axis_name="s")

    @functools.partial(
        pl.kernel, mesh=mesh,
        compiler_params=pltpu.CompilerParams(
            needs_layout_passes=False, use_tc_tiling_on_sc=False),
        out_type=jax.ShapeDtypeStruct((B, CANVAS_W), jnp.float32),
        scratch_types=[
            pltpu.VMEM((NSPL, NT, SPT), jnp.int32),
            pltpu.VMEM((NSPL, SPT), jnp.float32),
            pltpu.VMEM((CHUNK * CANVAS_W,), jnp.float32),
        ],
    )
    def sc_scatter(idx_hbm, w_hbm, out_hbm, idx_v, w_v, canvas_v):
        wid = lax.axis_index("s") * NCORES + lax.axis_index("c")
        lane_off = lax.iota(jnp.int32, 16) * CANVAS_W
        zeros16 = jnp.zeros((16,), jnp.float32)

        pltpu.sync_copy(idx_hbm.at[:, :, pl.ds(wid * SPT, SPT)], idx_v)
        pltpu.sync_copy(w_hbm.at[:, pl.ds(wid * SPT, SPT)], w_v)

        def chunk_body(k, carry):
            def zero_body(i, c):
                canvas_v[pl.ds(i * 16, 16)] = zeros16
                return c
            lax.fori_loop(0, CHUNK * CANVAS_W // 16, zero_body, 0)

            for sp in range(NSPL):
                wv = w_v[sp, pl.ds(k * CHUNK, CHUNK)]

                def t_body(t, c):
                    iv = idx_v[sp, t, pl.ds(k * CHUNK, CHUNK)]
                    plsc.addupdate_scatter(canvas_v, [iv + lane_off], wv)
                    return c
                lax.fori_loop(0, NT, t_body, 0)

            for l in range(CHUNK):
                pltpu.sync_copy(
                    canvas_v.at[pl.ds(l * CANVAS_W, CANVAS_W)],
                    out_hbm.at[wid * SPT + k * CHUNK + l])
            return carry

        lax.fori_loop(0, NCHUNKS, chunk_body, 0)

    return sc_scatter


_sc_scatter = _make_sc_scatter()


def kernel(x, W_lin, b_lin):
    wt = W_lin.T
    wperm = wt.reshape(NSPL, 7, DIN).transpose(1, 0, 2).reshape(DOUT, DIN)
    wcat = jnp.concatenate([wt, wperm], axis=0)
    bperm = b_lin.reshape(NSPL, 7).T.reshape(DOUT)
    bcat = jnp.concatenate([b_lin, bperm], axis=0)[:, None]

    t = jnp.linspace(0.0, 1.0, NT)
    b0 = ((1 - t) ** 2).reshape(1, NT, 1)
    b1 = (2 * (1 - t) * t).reshape(1, NT, 1)
    b2 = (t ** 2).reshape(1, NT, 1)

    grid = B // SB
    sample_t, idx_t, wgt_t = pl.pallas_call(
        _tc_points_kernel,
        grid=(grid,),
        in_specs=[
            pl.BlockSpec((2 * DOUT, DIN), lambda i: (0, 0)),
            pl.BlockSpec((2 * DOUT, 1), lambda i: (0, 0)),
            pl.BlockSpec((SB, DIN), lambda i: (i, 0)),
            pl.BlockSpec((1, NT, 1), lambda i: (0, 0, 0)),
            pl.BlockSpec((1, NT, 1), lambda i: (0, 0, 0)),
            pl.BlockSpec((1, NT, 1), lambda i: (0, 0, 0)),
        ],
        out_specs=[
            pl.BlockSpec((DOUT, SB), lambda i: (0, i)),
            pl.BlockSpec((NSPL, NT, SB), lambda i: (0, 0, i)),
            pl.BlockSpec((NSPL, SB), lambda i: (0, i)),
        ],
        out_shape=[
            jax.ShapeDtypeStruct((DOUT, B), jnp.float32),
            jax.ShapeDtypeStruct((NSPL, NT, B), jnp.int32),
            jax.ShapeDtypeStruct((NSPL, B), jnp.float32),
        ],
    )(wcat, bcat, x, b0, b1, b2)

    e3 = _sc_scatter(idx_t, wgt_t).reshape(B, 32, 32)

    sketch, log_prob, entropy = pl.pallas_call(
        _tc_fold_kernel,
        grid=(grid,),
        in_specs=[pl.BlockSpec((SB, 32, 32), lambda i: (i, 0, 0))],
        out_specs=[
            pl.BlockSpec((SB, 28, 28), lambda i: (i, 0, 0)),
            pl.BlockSpec((SB,), lambda i: (i,)),
            pl.BlockSpec((SB,), lambda i: (i,)),
        ],
        out_shape=[
            jax.ShapeDtypeStruct((B, 28, 28), jnp.float32),
            jax.ShapeDtypeStruct((B,), jnp.float32),
            jax.ShapeDtypeStruct((B,), jnp.float32),
        ],
    )(e3)

    return (sketch, log_prob, entropy, sample_t.T)


# speedup vs baseline: 126.5734x; 1.1222x over previous
import functools

import numpy as np
import jax
import jax.numpy as jnp
from jax import lax
from jax.experimental import pallas as pl
from jax.experimental.pallas import tpu as pltpu
from jax.experimental.pallas import tpu_sc as plsc

B = 4096
DIN = 128
DOUT = 112
NSPL = 16
NT = 50
SB = 256
CANVAS_W = 1024
NCORES = 2
NSUB = 16
NWORK = NCORES * NSUB
SPT = B // NWORK
CHUNK = 16
NCHUNKS = SPT // CHUNK

_LOG2PI = float(np.log(2.0 * np.pi))
ENTROPY_C = float(DOUT * (0.5 + 0.5 * _LOG2PI))
LOGPROB_C = float(DOUT * (-0.5 * _LOG2PI))


def _tc_points_kernel(w_ref, b_ref, x_ref, b0_ref, b1_ref, b2_ref,
                      sample_ref, idx_ref, wgt_ref):
    xb = x_ref[...].astype(jnp.bfloat16)
    mu = lax.dot_general(w_ref[...].astype(jnp.bfloat16), xb,
                         (((1,), (1,)), ((), ())),
                         preferred_element_type=jnp.float32)
    mu = mu + b_ref[...]
    sg = 1.0 / (1.0 + jnp.exp(-mu))
    sample_ref[...] = sg[0:DOUT]
    par = sg[DOUT:2 * DOUT] * 28.0
    p0x = par[0:16]
    p0y = par[16:32]
    p1x = par[32:48]
    p1y = par[48:64]
    p2x = par[64:80]
    p2y = par[80:96]
    wgt_ref[...] = par[96:112] * (-0.003)
    b0 = b0_ref[...]
    b1 = b1_ref[...]
    b2 = b2_ref[...]
    px = (b0 * p0x[:, None, :] + b1 * p1x[:, None, :]) + b2 * p2x[:, None, :]
    py = (b0 * p0y[:, None, :] + b1 * p1y[:, None, :]) + b2 * p2y[:, None, :]
    cx = jnp.round(px)
    cy = jnp.round(py)
    lane = lax.broadcasted_iota(jnp.int32, (1, 1, SB), 2)
    off = jnp.bitwise_and(lane, CHUNK - 1) * CANVAS_W
    idx_ref[...] = (cx * 32.0 + cy).astype(jnp.int32) + off


def _tc_fold_kernel(e_ref, sk_ref, lp_ref, en_ref):
    e = e_ref[...]
    ly = jnp.concatenate([e[:, :, 0:1], e[:, :, 0:27]], axis=2)
    ry = jnp.concatenate([e[:, :, 1:28],
                          e[:, :, 27:28] + 3.0 * e[:, :, 28:29]], axis=2)
    ty = (ly + e[:, :, 0:28]) + ry
    lx = jnp.concatenate([ty[:, 0:1], ty[:, 0:27]], axis=1)
    rx = jnp.concatenate([ty[:, 1:28],
                          ty[:, 27:28] + 3.0 * ty[:, 28:29]], axis=1)
    tx = (lx + ty[:, 0:28]) + rx
    sk_ref[...] = jnp.clip(tx + 0.3, 0.0, 1.0)
    lp_ref[...] = jnp.full((SB,), LOGPROB_C, jnp.float32)
    en_ref[...] = jnp.full((SB,), ENTROPY_C, jnp.float32)


def _make_sc_scatter():
    mesh = plsc.VectorSubcoreMesh(core_axis_name="c", subcore_axis_name="s")

    @functools.partial(
        pl.kernel, mesh=mesh,
        compiler_params=pltpu.CompilerParams(
            needs_layout_passes=False, use_tc_tiling_on_sc=False),
        out_type=jax.ShapeDtypeStruct((B, CANVAS_W), jnp.float32),
        scratch_types=[
            pltpu.VMEM((NSPL, NT, SPT), jnp.int32),
            pltpu.VMEM((NSPL, SPT), jnp.float32),
            pltpu.VMEM((CHUNK * CANVAS_W,), jnp.float32),
        ],
    )
    def sc_scatter(idx_hbm, w_hbm, out_hbm, idx_v, w_v, canvas_v):
        wid = lax.axis_index("s") * NCORES + lax.axis_index("c")
        zeros16 = jnp.zeros((16,), jnp.float32)

        pltpu.sync_copy(idx_hbm.at[:, :, pl.ds(wid * SPT, SPT)], idx_v)
        pltpu.sync_copy(w_hbm.at[:, pl.ds(wid * SPT, SPT)], w_v)

        def chunk_body(k, carry):
            def zero_body(i, c):
                canvas_v[pl.ds(i * 16, 16)] = zeros16
                return c
            lax.fori_loop(0, CHUNK * CANVAS_W // 16, zero_body, 0,
                          unroll=8)

            wvs = [w_v[sp, pl.ds(k * CHUNK, CHUNK)] for sp in range(NSPL)]

            def t_body(t, c):
                for sp in range(NSPL):
                    iv = idx_v[sp, t, pl.ds(k * CHUNK, CHUNK)]
                    plsc.addupdate_scatter(canvas_v, [iv], wvs[sp])
                return c
            lax.fori_loop(0, NT, t_body, 0)

            for l in range(CHUNK):
                pltpu.sync_copy(
                    canvas_v.at[pl.ds(l * CANVAS_W, CANVAS_W)],
                    out_hbm.at[wid * SPT + k * CHUNK + l])
            return carry

        lax.fori_loop(0, NCHUNKS, chunk_body, 0)

    return sc_scatter


_sc_scatter = _make_sc_scatter()


def kernel(x, W_lin, b_lin):
    wt = W_lin.T
    wperm = wt.reshape(NSPL, 7, DIN).transpose(1, 0, 2).reshape(DOUT, DIN)
    wcat = jnp.concatenate([wt, wperm], axis=0)
    bperm = b_lin.reshape(NSPL, 7).T.reshape(DOUT)
    bcat = jnp.concatenate([b_lin, bperm], axis=0)[:, None]

    t = jnp.linspace(0.0, 1.0, NT)
    b0 = ((1 - t) ** 2).reshape(1, NT, 1)
    b1 = (2 * (1 - t) * t).reshape(1, NT, 1)
    b2 = (t ** 2).reshape(1, NT, 1)

    grid = B // SB
    sample_t, idx_t, wgt_t = pl.pallas_call(
        _tc_points_kernel,
        grid=(grid,),
        in_specs=[
            pl.BlockSpec((2 * DOUT, DIN), lambda i: (0, 0)),
            pl.BlockSpec((2 * DOUT, 1), lambda i: (0, 0)),
            pl.BlockSpec((SB, DIN), lambda i: (i, 0)),
            pl.BlockSpec((1, NT, 1), lambda i: (0, 0, 0)),
            pl.BlockSpec((1, NT, 1), lambda i: (0, 0, 0)),
            pl.BlockSpec((1, NT, 1), lambda i: (0, 0, 0)),
        ],
        out_specs=[
            pl.BlockSpec((DOUT, SB), lambda i: (0, i)),
            pl.BlockSpec((NSPL, NT, SB), lambda i: (0, 0, i)),
            pl.BlockSpec((NSPL, SB), lambda i: (0, i)),
        ],
        out_shape=[
            jax.ShapeDtypeStruct((DOUT, B), jnp.float32),
            jax.ShapeDtypeStruct((NSPL, NT, B), jnp.int32),
            jax.ShapeDtypeStruct((NSPL, B), jnp.float32),
        ],
    )(wcat, bcat, x, b0, b1, b2)

    e3 = _sc_scatter(idx_t, wgt_t).reshape(B, 32, 32)

    sketch, log_prob, entropy = pl.pallas_call(
        _tc_fold_kernel,
        grid=(grid,),
        in_specs=[pl.BlockSpec((SB, 32, 32), lambda i: (i, 0, 0))],
        out_specs=[
            pl.BlockSpec((SB, 28, 28), lambda i: (i, 0, 0)),
            pl.BlockSpec((SB,), lambda i: (i,)),
            pl.BlockSpec((SB,), lambda i: (i,)),
        ],
        out_shape=[
            jax.ShapeDtypeStruct((B, 28, 28), jnp.float32),
            jax.ShapeDtypeStruct((B,), jnp.float32),
            jax.ShapeDtypeStruct((B,), jnp.float32),
        ],
    )(e3)

    return (sketch, log_prob, entropy, sample_t.T)


# speedup vs baseline: 131.7832x; 1.0412x over previous
import functools

import numpy as np
import jax
import jax.numpy as jnp
from jax import lax
from jax.experimental import pallas as pl
from jax.experimental.pallas import tpu as pltpu
from jax.experimental.pallas import tpu_sc as plsc

B = 4096
DIN = 128
DOUT = 112
NSPL = 16
NT = 50
SB = 256
CANVAS_W = 1024
NCORES = 2
NSUB = 16
NWORK = NCORES * NSUB
SPT = B // NWORK
CHUNK = 16
NCHUNKS = SPT // CHUNK

_LOG2PI = float(np.log(2.0 * np.pi))
ENTROPY_C = float(DOUT * (0.5 + 0.5 * _LOG2PI))
LOGPROB_C = float(DOUT * (-0.5 * _LOG2PI))


def _tc_points_kernel(w_ref, b_ref, x_ref, b0_ref, b1_ref, b2_ref,
                      sample_ref, idx_ref, wgt_ref):
    xb = x_ref[...].astype(jnp.bfloat16)
    mu = lax.dot_general(w_ref[...].astype(jnp.bfloat16), xb,
                         (((1,), (1,)), ((), ())),
                         preferred_element_type=jnp.float32)
    mu = mu + b_ref[...]
    sg = 1.0 / (1.0 + jnp.exp(-mu))
    sample_ref[...] = sg[0:DOUT]
    par = sg[DOUT:2 * DOUT] * 28.0
    p0x = par[0:16]
    p0y = par[16:32]
    p1x = par[32:48]
    p1y = par[48:64]
    p2x = par[64:80]
    p2y = par[80:96]
    wgt_ref[...] = par[96:112] * (-0.003)
    b0 = b0_ref[...]
    b1 = b1_ref[...]
    b2 = b2_ref[...]
    px = (b0 * p0x[:, None, :] + b1 * p1x[:, None, :]) + b2 * p2x[:, None, :]
    py = (b0 * p0y[:, None, :] + b1 * p1y[:, None, :]) + b2 * p2y[:, None, :]
    cx = jnp.round(px)
    cy = jnp.round(py)
    lane = lax.broadcasted_iota(jnp.int32, (1, 1, SB), 2)
    off = jnp.bitwise_and(lane, CHUNK - 1) * CANVAS_W
    idx_ref[...] = (cx * 32.0 + cy).astype(jnp.int32) + off


def _tc_fold_kernel(e_ref, sk_ref, lp_ref, en_ref):
    e = e_ref[...]
    ly = jnp.concatenate([e[:, :, 0:1], e[:, :, 0:27]], axis=2)
    ry = jnp.concatenate([e[:, :, 1:28],
                          e[:, :, 27:28] + 3.0 * e[:, :, 28:29]], axis=2)
    ty = (ly + e[:, :, 0:28]) + ry
    lx = jnp.concatenate([ty[:, 0:1], ty[:, 0:27]], axis=1)
    rx = jnp.concatenate([ty[:, 1:28],
                          ty[:, 27:28] + 3.0 * ty[:, 28:29]], axis=1)
    tx = (lx + ty[:, 0:28]) + rx
    sk_ref[...] = jnp.clip(tx + 0.3, 0.0, 1.0)
    lp_ref[...] = jnp.full((SB,), LOGPROB_C, jnp.float32)
    en_ref[...] = jnp.full((SB,), ENTROPY_C, jnp.float32)


def _make_sc_scatter():
    mesh = plsc.VectorSubcoreMesh(core_axis_name="c", subcore_axis_name="s")

    @functools.partial(
        pl.kernel, mesh=mesh,
        compiler_params=pltpu.CompilerParams(
            needs_layout_passes=False, use_tc_tiling_on_sc=False),
        out_type=jax.ShapeDtypeStruct((B, CANVAS_W), jnp.float32),
        scratch_types=[
            pltpu.VMEM((2, NSPL, NT, CHUNK), jnp.int32),
            pltpu.VMEM((NSPL, SPT), jnp.float32),
            pltpu.VMEM((2, CHUNK * CANVAS_W), jnp.float32),
            pltpu.SemaphoreType.DMA((2,)),
            pltpu.SemaphoreType.DMA((2,)),
        ],
    )
    def sc_scatter(idx_hbm, w_hbm, out_hbm, idx_v, w_v, canvas_v,
                   idx_sem, out_sem):
        wid = lax.axis_index("s") * NCORES + lax.axis_index("c")
        s_base = wid * SPT
        zeros16 = jnp.zeros((16,), jnp.float32)

        pltpu.sync_copy(w_hbm.at[:, pl.ds(s_base, SPT)], w_v)

        def start_idx(k, buf):
            return pltpu.make_async_copy(
                idx_hbm.at[:, :, pl.ds(s_base + k * CHUNK, CHUNK)],
                idx_v.at[buf], idx_sem.at[buf])

        start_idx(0, 0).start()
        out_handles = [None, None]

        for k in range(NCHUNKS):
            buf = k % 2
            start_idx(k, buf).wait()
            if k + 1 < NCHUNKS:
                start_idx(k + 1, 1 - buf).start()

            if out_handles[buf] is not None:
                for h in out_handles[buf]:
                    h.wait()
                out_handles[buf] = None

            cbuf = canvas_v.at[buf]

            def zero_body(i, c):
                cbuf[pl.ds(i * 16, 16)] = zeros16
                return c
            lax.fori_loop(0, CHUNK * CANVAS_W // 16, zero_body, 0,
                          unroll=8)

            wvs = [w_v[sp, pl.ds(k * CHUNK, CHUNK)] for sp in range(NSPL)]

            def t_body(t, c):
                for sp in range(NSPL):
                    iv = idx_v[buf, sp, t, :]
                    plsc.addupdate_scatter(cbuf, [iv], wvs[sp])
                return c
            lax.fori_loop(0, NT, t_body, 0)

            hs = []
            for l in range(CHUNK):
                h = pltpu.make_async_copy(
                    cbuf.at[pl.ds(l * CANVAS_W, CANVAS_W)],
                    out_hbm.at[s_base + k * CHUNK + l],
                    out_sem.at[buf])
                h.start()
                hs.append(h)
            out_handles[buf] = hs

        for hb in out_handles:
            if hb is not None:
                for h in hb:
                    h.wait()

    return sc_scatter


_sc_scatter = _make_sc_scatter()


def kernel(x, W_lin, b_lin):
    wt = W_lin.T
    wperm = wt.reshape(NSPL, 7, DIN).transpose(1, 0, 2).reshape(DOUT, DIN)
    wcat = jnp.concatenate([wt, wperm], axis=0)
    bperm = b_lin.reshape(NSPL, 7).T.reshape(DOUT)
    bcat = jnp.concatenate([b_lin, bperm], axis=0)[:, None]

    t = jnp.linspace(0.0, 1.0, NT)
    b0 = ((1 - t) ** 2).reshape(1, NT, 1)
    b1 = (2 * (1 - t) * t).reshape(1, NT, 1)
    b2 = (t ** 2).reshape(1, NT, 1)

    grid = B // SB
    sample_t, idx_t, wgt_t = pl.pallas_call(
        _tc_points_kernel,
        grid=(grid,),
        in_specs=[
            pl.BlockSpec((2 * DOUT, DIN), lambda i: (0, 0)),
            pl.BlockSpec((2 * DOUT, 1), lambda i: (0, 0)),
            pl.BlockSpec((SB, DIN), lambda i: (i, 0)),
            pl.BlockSpec((1, NT, 1), lambda i: (0, 0, 0)),
            pl.BlockSpec((1, NT, 1), lambda i: (0, 0, 0)),
            pl.BlockSpec((1, NT, 1), lambda i: (0, 0, 0)),
        ],
        out_specs=[
            pl.BlockSpec((DOUT, SB), lambda i: (0, i)),
            pl.BlockSpec((NSPL, NT, SB), lambda i: (0, 0, i)),
            pl.BlockSpec((NSPL, SB), lambda i: (0, i)),
        ],
        out_shape=[
            jax.ShapeDtypeStruct((DOUT, B), jnp.float32),
            jax.ShapeDtypeStruct((NSPL, NT, B), jnp.int32),
            jax.ShapeDtypeStruct((NSPL, B), jnp.float32),
        ],
    )(wcat, bcat, x, b0, b1, b2)

    e3 = _sc_scatter(idx_t, wgt_t).reshape(B, 32, 32)

    sketch, log_prob, entropy = pl.pallas_call(
        _tc_fold_kernel,
        grid=(grid,),
        in_specs=[pl.BlockSpec((SB, 32, 32), lambda i: (i, 0, 0))],
        out_specs=[
            pl.BlockSpec((SB, 28, 28), lambda i: (i, 0, 0)),
            pl.BlockSpec((SB,), lambda i: (i,)),
            pl.BlockSpec((SB,), lambda i: (i,)),
        ],
        out_shape=[
            jax.ShapeDtypeStruct((B, 28, 28), jnp.float32),
            jax.ShapeDtypeStruct((B,), jnp.float32),
            jax.ShapeDtypeStruct((B,), jnp.float32),
        ],
    )(e3)

    return (sketch, log_prob, entropy, sample_t.T)
